# Initial kernel scaffold; baseline (speedup 1.0000x reference)
import functools

import jax
import jax.numpy as jnp
from jax import lax
from jax.experimental import pallas as pl
from jax.experimental.pallas import tpu as pltpu
from jax.experimental.pallas import tpu_sc as plsc

VOCAB = 100000
EMBED_DIM = 64
BATCH = 16384
HIST = 50

NUM_CORES = 2
NUM_SUBCORES = 16
NUM_TILES = NUM_CORES * NUM_SUBCORES
LANES = 16
VPR = EMBED_DIM // LANES

BAGS_PER_TILE = BATCH // NUM_TILES
CHUNK_BAGS = 16
IDX_PER_CHUNK = CHUNK_BAGS * HIST
N_CHUNKS = BAGS_PER_TILE // CHUNK_BAGS
GATHER_SPLIT = 8
IDX_PER_GATHER = IDX_PER_CHUNK // GATHER_SPLIT


def _sc_body(ids_hbm, table_hbm, out_hbm, idx_v, rows_v, out_v, sem):
    wid = lax.axis_index("s") * NUM_CORES + lax.axis_index("c")
    base_bag = wid * BAGS_PER_TILE

    def chunk_body(ci, carry):
        bag_lo = base_bag + ci * CHUNK_BAGS
        pltpu.sync_copy(ids_hbm.at[pl.ds(bag_lo * HIST, IDX_PER_CHUNK)], idx_v)
        copies = []
        for g in range(GATHER_SPLIT):
            sl = pl.ds(g * IDX_PER_GATHER, IDX_PER_GATHER)
            copies.append(
                pltpu.async_copy(table_hbm.at[idx_v.at[sl]], rows_v.at[sl], sem))
        for c in copies:
            c.wait()

        def bag_body(r, carry2):
            def red_body(k, acc):
                row = r * HIST + k
                return tuple(
                    acc[j] + rows_v[row, pl.ds(j * LANES, LANES)]
                    for j in range(VPR))

            zero = jnp.zeros((LANES,), jnp.float32)
            acc = lax.fori_loop(0, HIST, red_body, (zero,) * VPR)
            for j in range(VPR):
                out_v[r, pl.ds(j * LANES, LANES)] = acc[j]
            return carry2

        lax.fori_loop(0, CHUNK_BAGS, bag_body, 0)
        pltpu.sync_copy(out_v, out_hbm.at[pl.ds(bag_lo, CHUNK_BAGS)])
        return carry

    lax.fori_loop(0, N_CHUNKS, chunk_body, 0)


@jax.jit
def kernel(ingredient_ids, embedding_table):
    ids_flat = ingredient_ids.reshape(-1).astype(jnp.int32)
    mesh = plsc.VectorSubcoreMesh(core_axis_name="c", subcore_axis_name="s")
    f = pl.kernel(
        _sc_body,
        mesh=mesh,
        out_type=jax.ShapeDtypeStruct((BATCH, EMBED_DIM), jnp.float32),
        scratch_types=[
            pltpu.VMEM((IDX_PER_CHUNK,), jnp.int32),
            pltpu.VMEM((IDX_PER_CHUNK, EMBED_DIM), jnp.float32),
            pltpu.VMEM((CHUNK_BAGS, EMBED_DIM), jnp.float32),
            pltpu.SemaphoreType.DMA,
        ],
    )
    return f(ids_flat, embedding_table)


# speedup vs baseline: 14.2333x; 14.2333x over previous
import functools

import jax
import jax.numpy as jnp
from jax import lax
from jax.experimental import pallas as pl
from jax.experimental.pallas import tpu as pltpu
from jax.experimental.pallas import tpu_sc as plsc

VOCAB = 100000
EMBED_DIM = 64
BATCH = 16384
HIST = 50

NUM_CORES = 2
NUM_SUBCORES = 16
NUM_TILES = NUM_CORES * NUM_SUBCORES
LANES = 16
VPR = EMBED_DIM // LANES

BAGS_PER_TILE = BATCH // NUM_TILES
CHUNK_BAGS = 16
IDX_PER_CHUNK = CHUNK_BAGS * HIST
N_CHUNKS = BAGS_PER_TILE // CHUNK_BAGS
GATHER_SPLIT = 10
IDX_PER_GATHER = IDX_PER_CHUNK // GATHER_SPLIT


def _sc_body(ids_hbm, table_hbm, out_hbm, idx_v, rows_v, out_v, sem):
    wid = lax.axis_index("s") * NUM_CORES + lax.axis_index("c")
    base_bag = wid * BAGS_PER_TILE

    def chunk_body(ci, carry):
        bag_lo = base_bag + ci * CHUNK_BAGS
        pltpu.sync_copy(ids_hbm.at[pl.ds(bag_lo * HIST, IDX_PER_CHUNK)], idx_v)
        copies = []
        for g in range(GATHER_SPLIT):
            sl = pl.ds(g * IDX_PER_GATHER, IDX_PER_GATHER)
            copies.append(
                pltpu.async_copy(table_hbm.at[idx_v.at[sl]], rows_v.at[sl], sem))
        for c in copies:
            c.wait()

        def bag_body(r, carry2):
            def red_body(k, acc):
                row = r * HIST + k
                return tuple(
                    acc[j] + rows_v[row, pl.ds(j * LANES, LANES)]
                    for j in range(VPR))

            zero = jnp.zeros((LANES,), jnp.float32)
            acc = lax.fori_loop(0, HIST, red_body, (zero,) * VPR)
            for j in range(VPR):
                out_v[r, pl.ds(j * LANES, LANES)] = acc[j]
            return carry2

        lax.fori_loop(0, CHUNK_BAGS, bag_body, 0)
        pltpu.sync_copy(out_v, out_hbm.at[pl.ds(bag_lo, CHUNK_BAGS)])
        return carry

    lax.fori_loop(0, N_CHUNKS, chunk_body, 0)


@jax.jit
def kernel(ingredient_ids, embedding_table):
    ids_flat = ingredient_ids.reshape(-1).astype(jnp.int32)
    mesh = plsc.VectorSubcoreMesh(core_axis_name="c", subcore_axis_name="s")
    f = pl.kernel(
        _sc_body,
        mesh=mesh,
        out_type=jax.ShapeDtypeStruct((BATCH, EMBED_DIM), jnp.float32),
        scratch_types=[
            pltpu.VMEM((IDX_PER_CHUNK,), jnp.int32),
            pltpu.VMEM((IDX_PER_CHUNK, EMBED_DIM), jnp.float32),
            pltpu.VMEM((CHUNK_BAGS, EMBED_DIM), jnp.float32),
            pltpu.SemaphoreType.DMA,
        ],
        compiler_params=pltpu.CompilerParams(use_tc_tiling_on_sc=False),
    )
    return f(ids_flat, embedding_table)


# speedup vs baseline: 22.0108x; 1.5464x over previous
import functools

import jax
import jax.numpy as jnp
from jax import lax
from jax.experimental import pallas as pl
from jax.experimental.pallas import tpu as pltpu
from jax.experimental.pallas import tpu_sc as plsc

VOCAB = 100000
EMBED_DIM = 64
BATCH = 16384
HIST = 50

NUM_CORES = 2
NUM_SUBCORES = 16
NUM_TILES = NUM_CORES * NUM_SUBCORES
LANES = 16
VPR = EMBED_DIM // LANES

BAGS_PER_TILE = BATCH // NUM_TILES
CHUNK_BAGS = 16
IDX_PER_CHUNK = CHUNK_BAGS * HIST
N_CHUNKS = BAGS_PER_TILE // CHUNK_BAGS
GATHER_SPLIT = 10
IDX_PER_GATHER = IDX_PER_CHUNK // GATHER_SPLIT


def _sc_body(ids_hbm, table_hbm, out_hbm,
             idx0, idx1, rows0, rows1, out0, out1,
             gsem0, gsem1, osem0, osem1):
    wid = lax.axis_index("s") * NUM_CORES + lax.axis_index("c")
    base_bag = wid * BAGS_PER_TILE
    idxs = (idx0, idx1)
    rows = (rows0, rows1)
    outs = (out0, out1)
    gsems = (gsem0, gsem1)
    osems = (osem0, osem1)

    def fire(ci, b):
        bag_lo = base_bag + ci * CHUNK_BAGS
        pltpu.sync_copy(ids_hbm.at[pl.ds(bag_lo * HIST, IDX_PER_CHUNK)], idxs[b])
        for g in range(GATHER_SPLIT):
            sl = pl.ds(g * IDX_PER_GATHER, IDX_PER_GATHER)
            pltpu.async_copy(table_hbm.at[idxs[b].at[sl]], rows[b].at[sl],
                             gsems[b])

    def drain_gather(b):
        pltpu.make_async_copy(table_hbm.at[pl.ds(0, IDX_PER_CHUNK)], rows[b],
                              gsems[b]).wait()

    def drain_out(b):
        pltpu.make_async_copy(outs[b], out_hbm.at[pl.ds(0, CHUNK_BAGS)],
                              osems[b]).wait()

    fire(0, 0)
    fire(1, 1)

    def outer(i, carry):
        for b in range(2):
            ci = 2 * i + b
            drain_gather(b)

            @pl.when(ci >= 2)
            def _():
                drain_out(b)

            def bag_body(r, carry2):
                def red_body(k, acc):
                    row = r * HIST + k
                    return tuple(
                        acc[j] + rows[b][row, pl.ds(j * LANES, LANES)]
                        for j in range(VPR))

                zero = jnp.zeros((LANES,), jnp.float32)
                acc = lax.fori_loop(0, HIST, red_body, (zero,) * VPR,
                                    unroll=10)
                for j in range(VPR):
                    outs[b][r, pl.ds(j * LANES, LANES)] = acc[j]
                return carry2

            lax.fori_loop(0, CHUNK_BAGS, bag_body, 0)
            pltpu.async_copy(
                outs[b],
                out_hbm.at[pl.ds(base_bag + ci * CHUNK_BAGS, CHUNK_BAGS)],
                osems[b])

            @pl.when(ci + 2 < N_CHUNKS)
            def _():
                fire(ci + 2, b)
        return carry

    lax.fori_loop(0, N_CHUNKS // 2, outer, 0)
    for b in range(2):
        drain_out(b)


@jax.jit
def kernel(ingredient_ids, embedding_table):
    ids_flat = ingredient_ids.reshape(-1).astype(jnp.int32)
    mesh = plsc.VectorSubcoreMesh(core_axis_name="c", subcore_axis_name="s")
    f = pl.kernel(
        _sc_body,
        mesh=mesh,
        out_type=jax.ShapeDtypeStruct((BATCH, EMBED_DIM), jnp.float32),
        scratch_types=[
            pltpu.VMEM((IDX_PER_CHUNK,), jnp.int32),
            pltpu.VMEM((IDX_PER_CHUNK,), jnp.int32),
            pltpu.VMEM((IDX_PER_CHUNK, EMBED_DIM), jnp.float32),
            pltpu.VMEM((IDX_PER_CHUNK, EMBED_DIM), jnp.float32),
            pltpu.VMEM((CHUNK_BAGS, EMBED_DIM), jnp.float32),
            pltpu.VMEM((CHUNK_BAGS, EMBED_DIM), jnp.float32),
            pltpu.SemaphoreType.DMA,
            pltpu.SemaphoreType.DMA,
            pltpu.SemaphoreType.DMA,
            pltpu.SemaphoreType.DMA,
        ],
        compiler_params=pltpu.CompilerParams(use_tc_tiling_on_sc=False),
    )
    return f(ids_flat, embedding_table)
